# baseline (speedup 1.0000x reference)
import functools

import numpy as np
import jax
import jax.numpy as jnp
from jax import lax
from jax.experimental import pallas as pl
from jax.experimental.pallas import tpu as pltpu
from jax.experimental.pallas import tpu_sc as plsc

B = 32
LATENT = 8
NPTS = 100000
XSIZE = 256
NFREQ = XSIZE // 2 + 1

NC = 2
NS = 16
IMG_PIX = XSIZE * XSIZE

SC_CHUNK = 2048
NPAD = 102400
TC_CHUNK = 2048
HI = float(XSIZE - 1.001)


def _filter_mats():
    x = np.arange(11) - 5.0
    k = np.exp(-0.5 * (x / 1.5) ** 2)
    k = k / k.sum()
    Kv = np.zeros((XSIZE, XSIZE))
    for i in range(XSIZE):
        for t in range(11):
            j = i + t - 5
            if 0 <= j < XSIZE:
                Kv[i, j] = k[t]
    n = np.arange(XSIZE)
    f = np.arange(NFREQ)
    Wy = np.exp(-2j * np.pi * np.outer(n, n) / XSIZE)
    WxT = np.exp(-2j * np.pi * np.outer(n, f) / XSIZE)
    A = Wy @ Kv
    Bm = Kv @ WxT
    C = np.exp(2j * np.pi * np.outer(n, n) / XSIZE) / XSIZE
    w = np.ones(NFREQ)
    w[1:NFREQ - 1] = 2.0
    D = (w[:, None] * np.exp(2j * np.pi * np.outer(f, n) / XSIZE)) / XSIZE
    cvt = lambda m: (np.asarray(m.real, np.float32), np.asarray(m.imag, np.float32))
    return cvt(A) + cvt(Bm) + cvt(C) + cvt(D)


_AR, _AI, _BR, _BI, _CR, _CI, _DR, _DI = _filter_mats()


def _points_body(zx, zy, zz, zb, ct, w, rr, sh,
                 itl, itr, vtl, vtr, vbl, vbr):
    hp = jax.lax.Precision.HIGHEST
    zbb = zb[...]
    dx = jnp.dot(zx[...], zbb, preferred_element_type=jnp.float32, precision=hp)
    dy = jnp.dot(zy[...], zbb, preferred_element_type=jnp.float32, precision=hp)
    dz = jnp.dot(zz[...], zbb, preferred_element_type=jnp.float32, precision=hp)
    cx = ct[0:1, :] + dx
    cy = ct[1:2, :] + dy
    cz = ct[2:3, :] + dz
    r = rr[...]
    s = sh[...]
    crx = r[:, 0:1] * cx + r[:, 1:2] * cy + r[:, 2:3] * cz + s[:, 0:1]
    cry = r[:, 3:4] * cx + r[:, 4:5] * cy + r[:, 5:6] * cz + s[:, 1:2]
    px = jnp.clip(crx + XSIZE / 2.0, 0.0, HI)
    py = jnp.clip(cry + XSIZE / 2.0, 0.0, HI)
    x0 = px.astype(jnp.int32)
    y0 = py.astype(jnp.int32)
    fx = px - x0.astype(jnp.float32)
    fy = py - y0.astype(jnp.float32)
    boff = (lax.broadcasted_iota(jnp.int32, (B, 1), 0) & (NS - 1)) * IMG_PIX
    base = boff + y0 * XSIZE + x0
    itl[...] = base
    itr[...] = base + 1
    ww = w[...]
    gx = 1.0 - fx
    gy = 1.0 - fy
    vtl[...] = ww * gx * gy
    vtr[...] = ww * fx * gy
    vbl[...] = ww * gx * fy
    vbr[...] = ww * fx * fy


def _points_call(zx, zy, zz, zb, ct, w2, rr, sh):
    grid = NPAD // TC_CHUNK
    full = lambda shape: pl.BlockSpec(shape, lambda j: (0,) * len(shape))
    chunk = lambda lead: pl.BlockSpec((lead, TC_CHUNK), lambda j: (0, j))
    oshape = jax.ShapeDtypeStruct((B, NPAD), jnp.int32)
    vshape = jax.ShapeDtypeStruct((B, NPAD), jnp.float32)
    return pl.pallas_call(
        _points_body,
        grid=(grid,),
        in_specs=[full((B, LATENT))] * 3 + [chunk(LATENT), chunk(3), chunk(1),
                                            full((B, 6)), full((B, 2))],
        out_specs=[chunk(B)] * 6,
        out_shape=[oshape] * 2 + [vshape] * 4,
    )(zx, zy, zz, zb, ct, w2, rr, sh)


NCHUNK = NPAD // SC_CHUNK


CORNER_OFF = (0, 0, XSIZE, XSIZE)
ACC_VIEW = NS * IMG_PIX - XSIZE


def _scatter_body(itl, itr, vtl, vtr, vbl, vbr, out_hbm, *scr):
    bufs = scr[:16]
    zero_v = scr[16]
    acc = scr[17]
    sem_in = scr[18:20]
    sem_sc = scr[20]
    s = lax.axis_index("s")
    c = lax.axis_index("c")
    b = c * NS + s
    my_off = s * IMG_PIX

    irefs = (itl, itr, itl, itr)
    vrefs = (vtl, vtr, vbl, vbr)

    def bufset(p):
        return bufs[8 * p:8 * p + 4], bufs[8 * p + 4:8 * p + 8]

    def issue_in(g, p):
        base = pl.multiple_of(g * SC_CHUNK, SC_CHUNK)
        ib, vb = bufset(p)
        hs = []
        for q in range(4):
            hs.append(pltpu.async_copy(irefs[q].at[b, pl.ds(base, SC_CHUNK)],
                                       ib[q], sem_in[p]))
            hs.append(pltpu.async_copy(vrefs[q].at[b, pl.ds(base, SC_CHUNK)],
                                       vb[q], sem_in[p]))
        return hs

    hin = issue_in(0, 0)

    def zinit(i, _):
        zero_v[pl.ds(i * 16, 16)] = jnp.zeros((16,), jnp.float32)
        return 0
    lax.fori_loop(0, 128, zinit, 0)

    def zcpy(i, _):
        off = pl.multiple_of(my_off + i * 2048, 2048)
        pltpu.sync_copy(zero_v, acc.at[pl.ds(off, 2048)])
        return 0
    lax.fori_loop(0, IMG_PIX // 2048, zcpy, 0)

    hsc = []
    for g in range(NCHUNK):
        p = g & 1
        for h in hsc:
            h.wait()
        nxt = issue_in(g + 1, (g + 1) & 1) if g + 1 < NCHUNK else []
        for h in hin:
            h.wait()
        ib, vb = bufset(p)
        hsc = [pltpu.async_copy(
                   vb[q],
                   acc.at[pl.ds(CORNER_OFF[q], ACC_VIEW)].at[ib[q]],
                   sem_sc, add=True)
               for q in range(4)]
        hin = nxt
    for h in hsc:
        h.wait()

    pltpu.sync_copy(acc.at[pl.ds(pl.multiple_of(my_off, IMG_PIX), IMG_PIX)],
                    out_hbm.at[b])


def _scatter_call(itl, itr, vtl, vtr, vbl, vbr):
    mesh = plsc.VectorSubcoreMesh(core_axis_name="c", subcore_axis_name="s")
    scratch = ([pltpu.VMEM((SC_CHUNK,), jnp.int32)] * 4 +
               [pltpu.VMEM((SC_CHUNK,), jnp.float32)] * 4) * 2
    scratch += [
        pltpu.VMEM((2048,), jnp.float32),
        pltpu.VMEM_SHARED((NS * IMG_PIX,), jnp.float32),
        pltpu.SemaphoreType.DMA,
        pltpu.SemaphoreType.DMA,
        pltpu.SemaphoreType.DMA,
    ]
    f = pl.kernel(
        _scatter_body,
        mesh=mesh,
        out_type=jax.ShapeDtypeStruct((B, IMG_PIX), jnp.float32),
        scratch_types=scratch,
    )
    return f(itl, itr, vtl, vtr, vbl, vbr)


NB = 8


def _filter_body(img_ref, ctf_ref, ar, ai, br, bi, cr, ci, dr, di, out_ref):
    hp = jax.lax.Precision.HIGHEST
    dot = functools.partial(jnp.dot, preferred_element_type=jnp.float32,
                            precision=hp)
    img = img_ref[...].reshape(NB * XSIZE, XSIZE)
    pr = dot(img, br[...])
    pi = dot(img, bi[...])
    arr = ar[...]
    aii = ai[...]
    grs = []
    gis = []
    for i in range(NB):
        pri = pr[i * XSIZE:(i + 1) * XSIZE]
        pii = pi[i * XSIZE:(i + 1) * XSIZE]
        fr = dot(arr, pri) - dot(aii, pii)
        fi = dot(arr, pii) + dot(aii, pri)
        ctf = ctf_ref[i]
        grs.append(ctf * fr)
        gis.append(ctf * fi)
    GR = jnp.concatenate(grs, axis=0)
    GI = jnp.concatenate(gis, axis=0)
    drr = dr[...]
    dii = di[...]
    QR = dot(GR, drr) - dot(GI, dii)
    QI = dot(GR, dii) + dot(GI, drr)
    crr = cr[...]
    cii = ci[...]
    for i in range(NB):
        qr = QR[i * XSIZE:(i + 1) * XSIZE]
        qi = QI[i * XSIZE:(i + 1) * XSIZE]
        out_ref[i] = dot(crr, qr) - dot(cii, qi)


def _filter_call(img, ctf):
    full = lambda shape: pl.BlockSpec(shape, lambda j: (0,) * len(shape))
    return pl.pallas_call(
        _filter_body,
        grid=(B // NB,),
        in_specs=[pl.BlockSpec((NB, XSIZE, XSIZE), lambda j: (j, 0, 0)),
                  pl.BlockSpec((NB, XSIZE, NFREQ), lambda j: (j, 0, 0)),
                  full((XSIZE, XSIZE)), full((XSIZE, XSIZE)),
                  full((XSIZE, NFREQ)), full((XSIZE, NFREQ)),
                  full((XSIZE, XSIZE)), full((XSIZE, XSIZE)),
                  full((NFREQ, XSIZE)), full((NFREQ, XSIZE))],
        out_specs=pl.BlockSpec((NB, XSIZE, XSIZE), lambda j: (j, 0, 0)),
        out_shape=jax.ShapeDtypeStruct((B, XSIZE, XSIZE), jnp.float32),
    )(img, ctf, _AR, _AI, _BR, _BI, _CR, _CI, _DR, _DI)


def kernel(z_x, z_y, z_z, Z_basis, coords, weights, R, shifts, ctf):
    pad = NPAD - NPTS
    zb = jnp.pad(Z_basis, ((0, 0), (0, pad)))
    ct = jnp.pad(coords.T, ((0, 0), (0, pad)))
    w2 = jnp.pad(weights, (0, pad)).reshape(1, NPAD)
    rr = R[:, :2, :].reshape(B, 6)
    outs = _points_call(z_x, z_y, z_z, zb, ct, w2, rr, shifts)
    img = _scatter_call(*outs)
    return _filter_call(img.reshape(B, XSIZE, XSIZE), ctf)


# speedup vs baseline: 1.0066x; 1.0066x over previous
import functools

import numpy as np
import jax
import jax.numpy as jnp
from jax import lax
from jax.experimental import pallas as pl
from jax.experimental.pallas import tpu as pltpu
from jax.experimental.pallas import tpu_sc as plsc

B = 32
LATENT = 8
NPTS = 100000
XSIZE = 256
NFREQ = XSIZE // 2 + 1

NC = 2
NS = 16
IMG_PIX = XSIZE * XSIZE

SC_CHUNK = 2048
NPAD = 102400
TC_CHUNK = 2048
HI = float(XSIZE - 1.001)


def _filter_mats():
    x = np.arange(11) - 5.0
    k = np.exp(-0.5 * (x / 1.5) ** 2)
    k = k / k.sum()
    Kv = np.zeros((XSIZE, XSIZE))
    for i in range(XSIZE):
        for t in range(11):
            j = i + t - 5
            if 0 <= j < XSIZE:
                Kv[i, j] = k[t]
    n = np.arange(XSIZE)
    f = np.arange(NFREQ)
    Wy = np.exp(-2j * np.pi * np.outer(n, n) / XSIZE)
    WxT = np.exp(-2j * np.pi * np.outer(n, f) / XSIZE)
    A = Wy @ Kv
    Bm = Kv @ WxT
    C = np.exp(2j * np.pi * np.outer(n, n) / XSIZE) / XSIZE
    w = np.ones(NFREQ)
    w[1:NFREQ - 1] = 2.0
    D = (w[:, None] * np.exp(2j * np.pi * np.outer(f, n) / XSIZE)) / XSIZE
    cvt = lambda m: (np.asarray(m.real, np.float32), np.asarray(m.imag, np.float32))
    return cvt(A) + cvt(Bm) + cvt(C) + cvt(D)


_AR, _AI, _BR, _BI, _CR, _CI, _DR, _DI = _filter_mats()


def _points_body(zx, zy, zz, zb, ct, w, rr, sh,
                 itl, itr, vtl, vtr, vbl, vbr):
    hp = jax.lax.Precision.HIGHEST
    zbb = zb[...]
    dx = jnp.dot(zx[...], zbb, preferred_element_type=jnp.float32, precision=hp)
    dy = jnp.dot(zy[...], zbb, preferred_element_type=jnp.float32, precision=hp)
    dz = jnp.dot(zz[...], zbb, preferred_element_type=jnp.float32, precision=hp)
    cx = ct[0:1, :] + dx
    cy = ct[1:2, :] + dy
    cz = ct[2:3, :] + dz
    r = rr[...]
    s = sh[...]
    crx = r[:, 0:1] * cx + r[:, 1:2] * cy + r[:, 2:3] * cz + s[:, 0:1]
    cry = r[:, 3:4] * cx + r[:, 4:5] * cy + r[:, 5:6] * cz + s[:, 1:2]
    px = jnp.clip(crx + XSIZE / 2.0, 0.0, HI)
    py = jnp.clip(cry + XSIZE / 2.0, 0.0, HI)
    x0 = px.astype(jnp.int32)
    y0 = py.astype(jnp.int32)
    fx = px - x0.astype(jnp.float32)
    fy = py - y0.astype(jnp.float32)
    boff = (lax.broadcasted_iota(jnp.int32, (B, 1), 0) & (NS - 1)) * IMG_PIX
    base = boff + y0 * XSIZE + x0
    itl[...] = base
    itr[...] = base + 1
    ww = w[...]
    gx = 1.0 - fx
    gy = 1.0 - fy
    vtl[...] = ww * gx * gy
    vtr[...] = ww * fx * gy
    vbl[...] = ww * gx * fy
    vbr[...] = ww * fx * fy


def _points_call(zx, zy, zz, zb, ct, w2, rr, sh):
    grid = NPAD // TC_CHUNK
    full = lambda shape: pl.BlockSpec(shape, lambda j: (0,) * len(shape))
    chunk = lambda lead: pl.BlockSpec((lead, TC_CHUNK), lambda j: (0, j))
    oshape = jax.ShapeDtypeStruct((B, NPAD), jnp.int32)
    vshape = jax.ShapeDtypeStruct((B, NPAD), jnp.float32)
    return pl.pallas_call(
        _points_body,
        grid=(grid,),
        in_specs=[full((B, LATENT))] * 3 + [chunk(LATENT), chunk(3), chunk(1),
                                            full((B, 6)), full((B, 2))],
        out_specs=[chunk(B)] * 6,
        out_shape=[oshape] * 2 + [vshape] * 4,
    )(zx, zy, zz, zb, ct, w2, rr, sh)


NCHUNK = NPAD // SC_CHUNK


CORNER_OFF = (0, 0, XSIZE, XSIZE)
ACC_VIEW = NS * IMG_PIX - XSIZE


def _scatter_body(itl, itr, vtl, vtr, vbl, vbr, out_hbm, *scr):
    bufs = scr[:16]
    zero_v = scr[16]
    acc = scr[17]
    sem_in = scr[18:20]
    sem_sc = scr[20]
    s = lax.axis_index("s")
    c = lax.axis_index("c")
    b = c * NS + s
    my_off = s * IMG_PIX

    irefs = (itl, itr, itl, itr)
    vrefs = (vtl, vtr, vbl, vbr)

    def bufset(p):
        return bufs[8 * p:8 * p + 4], bufs[8 * p + 4:8 * p + 8]

    def issue_in(g, p):
        base = pl.multiple_of(g * SC_CHUNK, SC_CHUNK)
        ib, vb = bufset(p)
        hs = []
        for q in range(4):
            hs.append(pltpu.async_copy(irefs[q].at[b, pl.ds(base, SC_CHUNK)],
                                       ib[q], sem_in[p]))
            hs.append(pltpu.async_copy(vrefs[q].at[b, pl.ds(base, SC_CHUNK)],
                                       vb[q], sem_in[p]))
        return hs

    hin = issue_in(0, 0)

    def zinit(i, _):
        zero_v[pl.ds(i * 16, 16)] = jnp.zeros((16,), jnp.float32)
        return 0
    lax.fori_loop(0, 128, zinit, 0)

    def zcpy(i, _):
        off = pl.multiple_of(my_off + i * 2048, 2048)
        pltpu.sync_copy(zero_v, acc.at[pl.ds(off, 2048)])
        return 0
    lax.fori_loop(0, IMG_PIX // 2048, zcpy, 0)

    hsc = []
    for g in range(NCHUNK):
        p = g & 1
        for h in hsc:
            h.wait()
        nxt = issue_in(g + 1, (g + 1) & 1) if g + 1 < NCHUNK else []
        for h in hin:
            h.wait()
        ib, vb = bufset(p)
        hsc = [pltpu.async_copy(
                   vb[q],
                   acc.at[pl.ds(CORNER_OFF[q], ACC_VIEW)].at[ib[q]],
                   sem_sc, add=True)
               for q in range(4)]
        hin = nxt
    for h in hsc:
        h.wait()

    pltpu.sync_copy(acc.at[pl.ds(pl.multiple_of(my_off, IMG_PIX), IMG_PIX)],
                    out_hbm.at[b])


def _scatter_call(itl, itr, vtl, vtr, vbl, vbr):
    mesh = plsc.VectorSubcoreMesh(core_axis_name="c", subcore_axis_name="s")
    scratch = ([pltpu.VMEM((SC_CHUNK,), jnp.int32)] * 4 +
               [pltpu.VMEM((SC_CHUNK,), jnp.float32)] * 4) * 2
    scratch += [
        pltpu.VMEM((2048,), jnp.float32),
        pltpu.VMEM_SHARED((NS * IMG_PIX,), jnp.float32),
        pltpu.SemaphoreType.DMA,
        pltpu.SemaphoreType.DMA,
        pltpu.SemaphoreType.DMA,
    ]
    f = pl.kernel(
        _scatter_body,
        mesh=mesh,
        out_type=jax.ShapeDtypeStruct((B, IMG_PIX), jnp.float32),
        scratch_types=scratch,
    )
    return f(itl, itr, vtl, vtr, vbl, vbr)


NB = 4


def _filter_body(img_ref, ctf_ref, ar, ai, br, bi, cr, ci, dr, di, out_ref):
    hp = jax.lax.Precision.HIGHEST
    dot = functools.partial(jnp.dot, preferred_element_type=jnp.float32,
                            precision=hp)
    img = img_ref[...].reshape(NB * XSIZE, XSIZE)
    pr = dot(img, br[...])
    pi = dot(img, bi[...])
    arr = ar[...]
    aii = ai[...]
    grs = []
    gis = []
    for i in range(NB):
        pri = pr[i * XSIZE:(i + 1) * XSIZE]
        pii = pi[i * XSIZE:(i + 1) * XSIZE]
        fr = dot(arr, pri) - dot(aii, pii)
        fi = dot(arr, pii) + dot(aii, pri)
        ctf = ctf_ref[i]
        grs.append(ctf * fr)
        gis.append(ctf * fi)
    GR = jnp.concatenate(grs, axis=0)
    GI = jnp.concatenate(gis, axis=0)
    drr = dr[...]
    dii = di[...]
    QR = dot(GR, drr) - dot(GI, dii)
    QI = dot(GR, dii) + dot(GI, drr)
    crr = cr[...]
    cii = ci[...]
    for i in range(NB):
        qr = QR[i * XSIZE:(i + 1) * XSIZE]
        qi = QI[i * XSIZE:(i + 1) * XSIZE]
        out_ref[i] = dot(crr, qr) - dot(cii, qi)


def _filter_call(img, ctf):
    full = lambda shape: pl.BlockSpec(shape, lambda j: (0,) * len(shape))
    return pl.pallas_call(
        _filter_body,
        grid=(B // NB,),
        in_specs=[pl.BlockSpec((NB, XSIZE, XSIZE), lambda j: (j, 0, 0)),
                  pl.BlockSpec((NB, XSIZE, NFREQ), lambda j: (j, 0, 0)),
                  full((XSIZE, XSIZE)), full((XSIZE, XSIZE)),
                  full((XSIZE, NFREQ)), full((XSIZE, NFREQ)),
                  full((XSIZE, XSIZE)), full((XSIZE, XSIZE)),
                  full((NFREQ, XSIZE)), full((NFREQ, XSIZE))],
        out_specs=pl.BlockSpec((NB, XSIZE, XSIZE), lambda j: (j, 0, 0)),
        out_shape=jax.ShapeDtypeStruct((B, XSIZE, XSIZE), jnp.float32),
    )(img, ctf, _AR, _AI, _BR, _BI, _CR, _CI, _DR, _DI)


def kernel(z_x, z_y, z_z, Z_basis, coords, weights, R, shifts, ctf):
    pad = NPAD - NPTS
    zb = jnp.pad(Z_basis, ((0, 0), (0, pad)))
    ct = jnp.pad(coords.T, ((0, 0), (0, pad)))
    w2 = jnp.pad(weights, (0, pad)).reshape(1, NPAD)
    rr = R[:, :2, :].reshape(B, 6)
    outs = _points_call(z_x, z_y, z_z, zb, ct, w2, rr, shifts)
    img = _scatter_call(*outs)
    return _filter_call(img.reshape(B, XSIZE, XSIZE), ctf)
